# Initial kernel scaffold; baseline (speedup 1.0000x reference)
import functools

import jax
import jax.numpy as jnp
from jax import lax
from jax.experimental import pallas as pl
from jax.experimental.pallas import tpu as pltpu
from jax.experimental.pallas import tpu_sc as plsc

_N = 64
_H = 512
_W = 512
_PH = 128
_PW = 128
_OFFSET = 120
_NC = 2
_NS = 16
_NW = _NC * _NS
_SPW = _N // _NW
_CH = 128


def _sc_body(batch_hbm, oy_hbm, ox_hbm, pat_hbm, out_hbm,
             buf, pbuf, offy_v, offx_v):
    wid = lax.axis_index("s") * _NC + lax.axis_index("c")
    for s in range(_SPW):
        n = wid * _SPW + s
        for c in range(_H // _CH):
            pltpu.sync_copy(batch_hbm.at[n, pl.ds(c * _CH, _CH), :], buf)
            pltpu.sync_copy(buf, out_hbm.at[n, pl.ds(c * _CH, _CH), :])
        pltpu.sync_copy(oy_hbm.at[n], offy_v)
        pltpu.sync_copy(ox_hbm.at[n], offx_v)
        oy = jnp.max(offy_v[...])
        ox = jnp.max(offx_v[...])
        pltpu.sync_copy(pat_hbm.at[n], pbuf)
        pltpu.sync_copy(pbuf, out_hbm.at[n, pl.ds(oy, _PH), pl.ds(ox, _PW)])


_mesh = plsc.VectorSubcoreMesh(
    core_axis_name="c", subcore_axis_name="s",
    num_cores=_NC, num_subcores=_NS)

_sc_kernel = pl.kernel(
    _sc_body,
    out_type=jax.ShapeDtypeStruct((_N, _H, _W), jnp.float32),
    mesh=_mesh,
    scratch_types=[
        pltpu.VMEM((_CH, _W), jnp.float32),
        pltpu.VMEM((_PH, _PW), jnp.float32),
        pltpu.VMEM((16,), jnp.int32),
        pltpu.VMEM((16,), jnp.int32),
    ],
)


def kernel(batch, translation, noise, ignore):
    off = translation - _OFFSET + noise
    off = off + jnp.array([[(_H - _PH) // 2, (_W - _PW) // 2]], jnp.float32)
    off = jnp.round(off).astype(jnp.int32)
    oy = jnp.clip(off[:, 0], 0, _H - _PH)
    ox = jnp.clip(off[:, 1], 0, _W - _PW)
    oy_b = jnp.broadcast_to(oy[:, None], (_N, 16))
    ox_b = jnp.broadcast_to(ox[:, None], (_N, 16))
    out = _sc_kernel(batch.reshape(_N, _H, _W), oy_b, ox_b,
                     ignore.reshape(_N, _PH, _PW))
    return out.reshape(_N, 1, _H, _W)


# speedup vs baseline: 23.4256x; 23.4256x over previous
import functools

import jax
import jax.numpy as jnp
from jax import lax
from jax.experimental import pallas as pl
from jax.experimental.pallas import tpu as pltpu
from jax.experimental.pallas import tpu_sc as plsc

_N = 64
_H = 512
_W = 512
_PH = 128
_PW = 128
_OFFSET = 120
_NC = 2
_NS = 16
_NW = _NC * _NS
_SPW = _N // _NW
_CH = 128


def _sc_body(batch_hbm, oy_hbm, ox_hbm, pat_hbm, out_hbm,
             buf, pbuf, offy_v, offx_v):
    wid = lax.axis_index("s") * _NC + lax.axis_index("c")
    for s in range(_SPW):
        n = wid * _SPW + s
        pltpu.sync_copy(oy_hbm.at[n], offy_v)
        pltpu.sync_copy(ox_hbm.at[n], offx_v)
        oy = offy_v[...][0]
        ox = offx_v[...][0]
        pltpu.sync_copy(pat_hbm.at[n], pbuf)
        for c in range(_H // _CH):
            base = c * _CH
            pltpu.sync_copy(batch_hbm.at[n, pl.ds(base, _CH), :], buf)
            lo = jnp.maximum(oy, base)
            hi = jnp.minimum(oy + _PH, base + _CH)

            def _row(r, _, base=base):
                br = r - base
                prow = r - oy
                for k in range(_PW // 16):
                    buf[br, pl.ds(ox + k * 16, 16)] = (
                        pbuf[prow, pl.ds(k * 16, 16)])
                return 0

            lax.fori_loop(lo, hi, _row, 0)
            pltpu.sync_copy(buf, out_hbm.at[n, pl.ds(base, _CH), :])


_mesh = plsc.VectorSubcoreMesh(
    core_axis_name="c", subcore_axis_name="s",
    num_cores=_NC, num_subcores=_NS)

_sc_kernel = pl.kernel(
    _sc_body,
    out_type=jax.ShapeDtypeStruct((_N, _H, _W), jnp.float32),
    mesh=_mesh,
    scratch_types=[
        pltpu.VMEM((_CH, _W), jnp.float32),
        pltpu.VMEM((_PH, _PW), jnp.float32),
        pltpu.VMEM((16,), jnp.int32),
        pltpu.VMEM((16,), jnp.int32),
    ],
    compiler_params=pltpu.CompilerParams(use_tc_tiling_on_sc=False),
)


def kernel(batch, translation, noise, ignore):
    off = translation - _OFFSET + noise
    off = off + jnp.array([[(_H - _PH) // 2, (_W - _PW) // 2]], jnp.float32)
    off = jnp.round(off).astype(jnp.int32)
    oy = jnp.clip(off[:, 0], 0, _H - _PH)
    ox = jnp.clip(off[:, 1], 0, _W - _PW)
    oy_b = jnp.broadcast_to(oy[:, None], (_N, 16))
    ox_b = jnp.broadcast_to(ox[:, None], (_N, 16))
    out = _sc_kernel(batch.reshape(_N, _H, _W), oy_b, ox_b,
                     ignore.reshape(_N, _PH, _PW))
    return out.reshape(_N, 1, _H, _W)


# speedup vs baseline: 24.5144x; 1.0465x over previous
import jax
import jax.numpy as jnp
from jax import lax
from jax.experimental import pallas as pl
from jax.experimental.pallas import tpu as pltpu
from jax.experimental.pallas import tpu_sc as plsc

_N = 64
_H = 512
_W = 512
_PH = 128
_PW = 128
_OFFSET = 120
_NC = 2
_NS = 16
_NW = _NC * _NS
_SPW = _N // _NW
_CH = 64
_UPS = _H // _CH


def _sc_body(batch_hbm, oy_hbm, ox_hbm, pat_hbm, out_hbm,
             buf0, buf1, pb0, pb1, offy_v, offx_v,
             rs0, rs1, ws0, ws1, ps0, ps1):
    wid = lax.axis_index("s") * _NC + lax.axis_index("c")
    n0 = wid * _SPW
    bufs = (buf0, buf1)
    rsems = (rs0, rs1)
    wsems = (ws0, ws1)
    pbufs = (pb0, pb1)
    psems = (ps0, ps1)

    oys, oxs, phandles = [], [], []
    for s in range(_SPW):
        pltpu.sync_copy(oy_hbm.at[n0 + s], offy_v)
        oys.append(offy_v[...][0])
        pltpu.sync_copy(ox_hbm.at[n0 + s], offx_v)
        oxs.append(offx_v[...][0])
        phandles.append(pltpu.async_copy(pat_hbm.at[n0 + s], pbufs[s], psems[s]))

    units = [(n0 + s, s, c * _CH) for s in range(_SPW) for c in range(_UPS)]
    nu = len(units)
    read_h = {0: pltpu.async_copy(
        batch_hbm.at[units[0][0], pl.ds(units[0][2], _CH), :], bufs[0], rsems[0])}
    write_h = {}
    for i, (n, s, base) in enumerate(units):
        b = i % 2
        if i + 1 < nu:
            n2, _, base2 = units[i + 1]
            if i >= 1:
                write_h[i - 1].wait()
            read_h[i + 1] = pltpu.async_copy(
                batch_hbm.at[n2, pl.ds(base2, _CH), :],
                bufs[(i + 1) % 2], rsems[(i + 1) % 2])
        read_h[i].wait()
        if i % _UPS == 0:
            phandles[s].wait()
        oy, ox = oys[s], oxs[s]
        pb = pbufs[s]
        buf = bufs[b]
        lo = jnp.maximum(oy, base)
        hi = jnp.minimum(oy + _PH, base + _CH)

        def _row(r, _, buf=buf, pb=pb, oy=oy, ox=ox, base=base):
            br = r - base
            prow = r - oy
            for k in range(_PW // 16):
                buf[br, pl.ds(ox + k * 16, 16)] = pb[prow, pl.ds(k * 16, 16)]
            return 0

        lax.fori_loop(lo, hi, _row, 0)
        write_h[i] = pltpu.async_copy(
            buf, out_hbm.at[n, pl.ds(base, _CH), :], wsems[b])
    write_h[nu - 2].wait()
    write_h[nu - 1].wait()


_mesh = plsc.VectorSubcoreMesh(
    core_axis_name="c", subcore_axis_name="s",
    num_cores=_NC, num_subcores=_NS)

_sc_kernel = pl.kernel(
    _sc_body,
    out_type=jax.ShapeDtypeStruct((_N, _H, _W), jnp.float32),
    mesh=_mesh,
    scratch_types=[
        pltpu.VMEM((_CH, _W), jnp.float32),
        pltpu.VMEM((_CH, _W), jnp.float32),
        pltpu.VMEM((_PH, _PW), jnp.float32),
        pltpu.VMEM((_PH, _PW), jnp.float32),
        pltpu.VMEM((16,), jnp.int32),
        pltpu.VMEM((16,), jnp.int32),
        pltpu.SemaphoreType.DMA,
        pltpu.SemaphoreType.DMA,
        pltpu.SemaphoreType.DMA,
        pltpu.SemaphoreType.DMA,
        pltpu.SemaphoreType.DMA,
        pltpu.SemaphoreType.DMA,
    ],
    compiler_params=pltpu.CompilerParams(use_tc_tiling_on_sc=False),
)


def kernel(batch, translation, noise, ignore):
    off = translation - _OFFSET + noise
    off = off + jnp.array([[(_H - _PH) // 2, (_W - _PW) // 2]], jnp.float32)
    off = jnp.round(off).astype(jnp.int32)
    oy = jnp.clip(off[:, 0], 0, _H - _PH)
    ox = jnp.clip(off[:, 1], 0, _W - _PW)
    oy_b = jnp.broadcast_to(oy[:, None], (_N, 16))
    ox_b = jnp.broadcast_to(ox[:, None], (_N, 16))
    out = _sc_kernel(batch.reshape(_N, _H, _W), oy_b, ox_b,
                     ignore.reshape(_N, _PH, _PW))
    return out.reshape(_N, 1, _H, _W)


# speedup vs baseline: 65.6439x; 2.6778x over previous
import jax
import jax.numpy as jnp
from jax import lax
from jax.experimental import pallas as pl
from jax.experimental.pallas import tpu as pltpu
from jax.experimental.pallas import tpu_sc as plsc

_N = 64
_H = 512
_W = 512
_PH = 128
_PW = 128
_OFFSET = 120
_NC = 2
_NS = 16
_NW = _NC * _NS
_SPW = _N // _NW
_CH = 64
_UPS = _H // _CH
_D = 2


def _sc_body(batch_hbm, off_hbm, pat_hbm, out_hbm,
             buf0, buf1, pb0, pb1, off_v,
             rs0, rs1, ws0, ws1, ps0, ps1):
    wid = lax.axis_index("s") * _NC + lax.axis_index("c")
    n0 = wid * _SPW
    bufs = (buf0, buf1)
    rsems = (rs0, rs1)
    wsems = (ws0, ws1)
    pbufs = (pb0, pb1)
    psems = (ps0, ps1)

    units = [(n0 + s, s, c * _CH) for s in range(_SPW) for c in range(_UPS)]
    nu = len(units)

    def _read(j):
        nj, _, basej = units[j]
        return pltpu.async_copy(
            batch_hbm.at[nj, pl.ds(basej, _CH), :], bufs[j % _D], rsems[j % _D])

    read_h = {j: _read(j) for j in range(_D - 1)}
    phandles = [pltpu.async_copy(pat_hbm.at[n0 + s], pbufs[s], psems[s])
                for s in range(_SPW)]
    pltpu.sync_copy(off_hbm.at[wid], off_v)
    oys = [off_v[s, pl.ds(0, 16)][0] for s in range(_SPW)]
    oxs = [off_v[s, pl.ds(16, 16)][0] for s in range(_SPW)]
    write_h = {}
    for i, (n, s, base) in enumerate(units):
        b = i % _D
        ahead = i + _D - 1
        if ahead < nu:
            if ahead - _D >= 0:
                write_h[ahead - _D].wait()
            read_h[ahead] = _read(ahead)
        read_h[i].wait()
        if i % _UPS == 0:
            phandles[s].wait()
        oy, ox = oys[s], oxs[s]
        pb = pbufs[s]
        buf = bufs[b]
        lo = jnp.maximum(oy, base)
        hi = jnp.minimum(oy + _PH, base + _CH)

        def _row(row, _, buf=buf, pb=pb, oy=oy, ox=ox, base=base):
            br = row - base
            prow = row - oy
            lane = lax.iota(jnp.int32, 16)
            q = ox // 16
            r = jnp.bitwise_and(ox, 15)
            rot = jnp.bitwise_and(lane - r, 15)
            head = lane < r
            nk = _PW // 16
            g = []
            for k in range(nk):
                pk = pb[prow, pl.ds(k * 16, 16)]
                g.append(lax.gather(
                    pk, rot[:, None],
                    lax.GatherDimensionNumbers(
                        offset_dims=(), collapsed_slice_dims=(0,),
                        start_index_map=(0,)),
                    slice_sizes=(1,),
                    mode=lax.GatherScatterMode.PROMISE_IN_BOUNDS))
            d0 = buf[br, pl.ds(16 * q, 16)]
            buf[br, pl.ds(16 * q, 16)] = jnp.where(head, d0, g[0])
            for k in range(1, nk):
                buf[br, pl.ds(16 * (q + k), 16)] = jnp.where(head, g[k - 1], g[k])
            o8 = jnp.minimum(16 * (q + nk), _W - 16)
            d8 = buf[br, pl.ds(o8, 16)]
            buf[br, pl.ds(o8, 16)] = jnp.where(head, g[nk - 1], d8)
            return 0

        lax.fori_loop(lo, hi, _row, 0)
        write_h[i] = pltpu.async_copy(
            buf, out_hbm.at[n, pl.ds(base, _CH), :], wsems[b])
    for j in range(max(0, nu - _D), nu):
        if j in write_h:
            write_h[j].wait()


_mesh = plsc.VectorSubcoreMesh(
    core_axis_name="c", subcore_axis_name="s",
    num_cores=_NC, num_subcores=_NS)

_sc_kernel = pl.kernel(
    _sc_body,
    out_type=jax.ShapeDtypeStruct((_N, _H, _W), jnp.float32),
    mesh=_mesh,
    scratch_types=(
        [pltpu.VMEM((_CH, _W), jnp.float32)] * _D
        + [pltpu.VMEM((_PH, _PW), jnp.float32)] * 2
        + [pltpu.VMEM((_SPW, 32), jnp.int32)]
        + [pltpu.SemaphoreType.DMA] * (2 * _D + 2)
    ),
)


def kernel(batch, translation, noise, ignore):
    off = translation - _OFFSET + noise
    off = off + jnp.array([[(_H - _PH) // 2, (_W - _PW) // 2]], jnp.float32)
    off = jnp.round(off).astype(jnp.int32)
    oy = jnp.clip(off[:, 0], 0, _H - _PH)
    ox = jnp.clip(off[:, 1], 0, _W - _PW)
    off_b = jnp.concatenate(
        [jnp.broadcast_to(oy[:, None, None], (_N, 1, 16)),
         jnp.broadcast_to(ox[:, None, None], (_N, 1, 16))], axis=2)
    off_b = off_b.reshape(_NW, _SPW, 32)
    out = _sc_kernel(batch.reshape(_N, _H, _W), off_b,
                     ignore.reshape(_N, _PH, _PW))
    return out.reshape(_N, 1, _H, _W)
